# baseline (speedup 1.0000x reference)
import jax
import jax.numpy as jnp
from jax import lax
from jax.experimental import pallas as pl
from jax.experimental.pallas import tpu as pltpu
from jax.experimental.pallas import tpu_sc as plsc

B = 16384
S = 200
ITEMS = 1000
CATS = 100
E = 32
NC, NS, L = 2, 16, 16
NW = NC * NS
RW = B // NW
CB = 64
NCHUNK = RW // CB
SP = S + 1
GJ = 128
NJ = RW // GJ


def _sc_body(hist, item, neg, clist, itab, zrow, zrow_c,
             counts_i, ohic, ohnc, it_i, ng_i,
             hbuf, cbuf, ohib, ohnb, catb, itemv, negv,
             idxb, ngxb, rowb, sem):
    wid = lax.axis_index("s") * NC + lax.axis_index("c")
    base = wid * RW
    iota = lax.iota(jnp.int32, L)
    ones = jnp.full((L,), 1.0, jnp.float32)
    neg_ones = jnp.full((L,), -1.0, jnp.float32)

    pltpu.sync_copy(clist, catb)
    pltpu.sync_copy(item.at[pl.ds(base, RW)], itemv)
    pltpu.sync_copy(neg.at[pl.ds(base, RW)], negv)

    for j in range(NJ):
        pltpu.sync_copy(item.at[pl.ds(base + j * GJ, GJ)], idxb.at[j])
        pltpu.sync_copy(neg.at[pl.ds(base + j * GJ, GJ)], ngxb.at[j])
    for j in range(NJ):
        pltpu.async_copy(itab.at[idxb.at[j]], rowb, sem).wait()
        pltpu.sync_copy(rowb, it_i.at[pl.ds(base + j * GJ, GJ)])
        pltpu.async_copy(itab.at[ngxb.at[j]], rowb, sem).wait()
        pltpu.sync_copy(rowb, ng_i.at[pl.ds(base + j * GJ, GJ)])

    pltpu.sync_copy(zrow, cbuf)

    for chunk in range(NCHUNK):
        cb = base + chunk * CB
        pltpu.sync_copy(hist.at[pl.ds(cb, CB)], hbuf)
        pltpu.sync_copy(zrow_c, ohib)
        pltpu.sync_copy(zrow_c, ohnb)

        rgs = [iota + rg * L for rg in range(CB // L)]

        def scatter_pass(val):
            def sbody(k, sv, val=val):
                for _ in range(4):
                    hs = [plsc.load_gather(hbuf, [rows, sv]) for rows in rgs]
                    for rows, h in zip(rgs, hs):
                        plsc.addupdate_scatter(cbuf, [rows, h], val)
                    sv = sv + 1
                return sv

            lax.fori_loop(0, S // 4, sbody, jnp.zeros((L,), jnp.int32))

        scatter_pass(ones)
        for rg in range(CB // L):
            rows = iota + rg * L
            iv = itemv[pl.ds(chunk * CB + rg * L, L)]
            plsc.addupdate_scatter(ohib, [rows, plsc.load_gather(catb, [iv])],
                                   ones)
            nv = negv[pl.ds(chunk * CB + rg * L, L)]
            plsc.addupdate_scatter(ohnb, [rows, plsc.load_gather(catb, [nv])],
                                   ones)
        pltpu.sync_copy(cbuf, counts_i.at[pl.ds(cb, CB)])
        pltpu.sync_copy(ohib, ohic.at[pl.ds(cb, CB)])
        pltpu.sync_copy(ohnb, ohnc.at[pl.ds(cb, CB)])
        if chunk != NCHUNK - 1:
            scatter_pass(neg_ones)


def _sc_counts(history_item, item, neg_item, category_list, item_table):
    f32 = jnp.float32
    history_item = jnp.pad(history_item, ((0, 0), (0, SP - S)))
    zrow = jnp.zeros((CB, ITEMS), f32)
    zrow_c = jnp.zeros((CB, CATS), f32)
    out_type = (
        jax.ShapeDtypeStruct((B, ITEMS), f32),
        jax.ShapeDtypeStruct((B, CATS), f32),
        jax.ShapeDtypeStruct((B, CATS), f32),
        jax.ShapeDtypeStruct((B, E), f32),
        jax.ShapeDtypeStruct((B, E), f32),
    )
    scratch = [
        pltpu.VMEM((CB, SP), jnp.int32),
        pltpu.VMEM((CB, ITEMS), f32),
        pltpu.VMEM((CB, CATS), f32),
        pltpu.VMEM((CB, CATS), f32),
        pltpu.VMEM((ITEMS,), jnp.int32),
        pltpu.VMEM((RW,), jnp.int32),
        pltpu.VMEM((RW,), jnp.int32),
        pltpu.VMEM((NJ, GJ), jnp.int32),
        pltpu.VMEM((NJ, GJ), jnp.int32),
        pltpu.VMEM((GJ, E), f32),
        pltpu.SemaphoreType.DMA,
    ]
    mesh = plsc.VectorSubcoreMesh(core_axis_name="c", subcore_axis_name="s")
    return pl.kernel(_sc_body, out_type=out_type, mesh=mesh,
                     scratch_types=scratch,
                     compiler_params=pltpu.CompilerParams(
                         use_tc_tiling_on_sc=False,
                         needs_layout_passes=False))(
        history_item, item, neg_item, category_list, item_table, zrow, zrow_c)


BM = 512
NB = B // BM


def _tc_body(ci_ref, ohi_ref, ohn_ref, iti_ref, ngi_ref,
             cl_ref, itab_ref, ctab_ref, wu_ref, bu_ref,
             g_ref, b_ref, w1_ref, b1_ref, w2_ref, b2_ref,
             wo_ref, bo_ref,
             out_ref, hall_ref, tab_ref, st_ref):
    p = pl.program_id(0)
    i = pl.program_id(1)
    f32 = jnp.float32
    hp = lax.Precision.HIGHEST

    @pl.when(p == 0)
    def _phase0():
        @pl.when(i == 0)
        def _():
            m = (cl_ref[...] == lax.broadcasted_iota(jnp.int32,
                                                     (ITEMS, CATS), 1))
            tab_ref[:, :E] = itab_ref[...]
            tab_ref[:, E:] = lax.dot(m.astype(f32), ctab_ref[...],
                                     preferred_element_type=f32, precision=hp)
            st_ref[...] = jnp.zeros_like(st_ref)

        h = lax.dot(ci_ref[...], tab_ref[...],
                    preferred_element_type=f32, precision=hp) * (1.0 / S)
        hall_ref[pl.ds(i * BM, BM), :] = h
        st_ref[0:1, :] += jnp.sum(h, axis=0, keepdims=True)
        st_ref[1:2, :] += jnp.sum(h * h, axis=0, keepdims=True)

    @pl.when(p == 1)
    def _phase1():
        inv_b = 1.0 / B
        eps = 1e-5
        mean = st_ref[0:1, :] * inv_b
        var = st_ref[1:2, :] * inv_b - mean * mean
        sc = g_ref[...] * lax.rsqrt(var + eps)
        sh = b_ref[...] - mean * sc

        bn = hall_ref[pl.ds(i * BM, BM), :] * sc + sh
        ue = lax.dot(bn, wu_ref[...]) + bu_ref[...]

        def head(e_i, oh_c):
            e_c = lax.dot(oh_c, ctab_ref[...], preferred_element_type=f32,
                          precision=hp)
            din = jnp.concatenate([ue, e_i, e_c], axis=1)
            t = jnp.maximum(lax.dot(din, w1_ref[...]) + b1_ref[...], 0.0)
            t = jnp.maximum(lax.dot(t, w2_ref[...]) + b2_ref[...], 0.0)
            return lax.dot(t, wo_ref[...]) + bo_ref[...]

        out_ref[...] = (head(iti_ref[...], ohi_ref[...])
                        - head(ngi_ref[...], ohn_ref[...]))


def _tc_head(counts_i, oh_it, oh_ng, it_i, ng_i, category_list,
             item_table, cat_tab100, W_user, b_user, gamma, beta,
             W1, b1, W2, b2, Wout, bout):
    f32 = jnp.float32

    def cmap(bs):
        return pl.BlockSpec(bs, lambda p, i: (i * (1 - p), 0))

    def emap(bs):
        return pl.BlockSpec(bs, lambda p, i: (i * p, 0))

    def wmap(shape):
        return pl.BlockSpec(shape, lambda p, i: (0, 0))

    in_specs = [
        cmap((BM, ITEMS)),
        emap((BM, CATS)), emap((BM, CATS)), emap((BM, E)), emap((BM, E)),
        wmap((ITEMS, 1)), wmap((ITEMS, E)), wmap((CATS, E)),
        wmap((64, 64)), wmap((1, 64)),
        wmap((1, 64)), wmap((1, 64)),
        wmap((128, 200)), wmap((1, 200)),
        wmap((200, 80)), wmap((1, 80)),
        wmap((80, 1)), wmap((1, 1)),
    ]
    return pl.pallas_call(
        _tc_body,
        grid=(2, NB),
        in_specs=in_specs,
        out_specs=pl.BlockSpec((BM, 1), lambda p, i: (i, 0)),
        out_shape=jax.ShapeDtypeStruct((B, 1), f32),
        scratch_shapes=[
            pltpu.VMEM((B, 64), f32),
            pltpu.VMEM((ITEMS, 64), f32),
            pltpu.VMEM((2, 64), f32),
        ],
    )(counts_i, oh_it, oh_ng, it_i, ng_i,
      category_list.reshape(ITEMS, 1), item_table, cat_tab100,
      W_user, b_user.reshape(1, 64),
      gamma.reshape(1, 64), beta.reshape(1, 64),
      W1, b1.reshape(1, 200),
      W2, b2.reshape(1, 80), Wout, bout.reshape(1, 1))


def kernel(user, item, neg_item, history_item, category_list, item_table,
           cat_table, W_user, b_user, gamma, beta, W1, b1, W2, b2, Wout, bout):
    counts_i, oh_it, oh_ng, it_i, ng_i = _sc_counts(
        history_item, item, neg_item, category_list, item_table)
    out = _tc_head(counts_i, oh_it, oh_ng, it_i, ng_i, category_list,
                   item_table, cat_table[:CATS], W_user, b_user, gamma, beta,
                   W1, b1, W2, b2, Wout, bout)
    return out[:, 0]


# speedup vs baseline: 1.1988x; 1.1988x over previous
import jax
import jax.numpy as jnp
from jax import lax
from jax.experimental import pallas as pl
from jax.experimental.pallas import tpu as pltpu
from jax.experimental.pallas import tpu_sc as plsc

B = 16384
S = 200
ITEMS = 1000
CATS = 100
E = 32
NC, NS, L = 2, 16, 16
NW = NC * NS
RW = B // NW
CB = 64
NCHUNK = RW // CB
GJ = 128
NJ = RW // GJ


def _sc_body(hist, item, neg, clist, itab, zrow, zrow_c,
             counts_i, ohic, ohnc, it_i, ng_i,
             hbuf, cbuf, ohib, ohnb, catb, itemv, negv,
             idxb, ngxb, rowb, sem):
    wid = lax.axis_index("s") * NC + lax.axis_index("c")
    base = wid * RW
    iota = lax.iota(jnp.int32, L)
    ones = jnp.full((L,), 1.0, jnp.float32)
    neg_ones = jnp.full((L,), -1.0, jnp.float32)

    pltpu.sync_copy(clist, catb)
    pltpu.sync_copy(item.at[pl.ds(base, RW)], itemv)
    pltpu.sync_copy(neg.at[pl.ds(base, RW)], negv)

    for j in range(NJ):
        pltpu.sync_copy(item.at[pl.ds(base + j * GJ, GJ)], idxb.at[j])
        pltpu.sync_copy(neg.at[pl.ds(base + j * GJ, GJ)], ngxb.at[j])
    for j in range(NJ):
        pltpu.async_copy(itab.at[idxb.at[j]], rowb, sem).wait()
        pltpu.sync_copy(rowb, it_i.at[pl.ds(base + j * GJ, GJ)])
        pltpu.async_copy(itab.at[ngxb.at[j]], rowb, sem).wait()
        pltpu.sync_copy(rowb, ng_i.at[pl.ds(base + j * GJ, GJ)])

    pltpu.sync_copy(zrow, cbuf)

    for chunk in range(NCHUNK):
        cb = base + chunk * CB
        pltpu.sync_copy(hist.at[pl.ds(cb, CB)], hbuf)
        pltpu.sync_copy(zrow_c, ohib)
        pltpu.sync_copy(zrow_c, ohnb)

        rgs = [iota + rg * L for rg in range(CB // L)]

        def scatter_pass(val):
            def sbody(k, sv, val=val):
                for _ in range(4):
                    hs = [plsc.load_gather(hbuf, [rows, sv]) for rows in rgs]
                    for rows, h in zip(rgs, hs):
                        plsc.addupdate_scatter(cbuf, [rows, h], val)
                    sv = sv + 1
                return sv

            lax.fori_loop(0, S // 4, sbody, jnp.zeros((L,), jnp.int32))

        scatter_pass(ones)
        for rg in range(CB // L):
            rows = iota + rg * L
            iv = itemv[pl.ds(chunk * CB + rg * L, L)]
            plsc.addupdate_scatter(ohib, [rows, plsc.load_gather(catb, [iv])],
                                   ones)
            nv = negv[pl.ds(chunk * CB + rg * L, L)]
            plsc.addupdate_scatter(ohnb, [rows, plsc.load_gather(catb, [nv])],
                                   ones)
        pltpu.sync_copy(cbuf, counts_i.at[pl.ds(cb, CB)])
        pltpu.sync_copy(ohib, ohic.at[pl.ds(cb, CB)])
        pltpu.sync_copy(ohnb, ohnc.at[pl.ds(cb, CB)])
        if chunk != NCHUNK - 1:
            scatter_pass(neg_ones)


def _sc_counts(history_item, item, neg_item, category_list, item_table):
    f32 = jnp.float32
    zrow = jnp.zeros((CB, ITEMS), f32)
    zrow_c = jnp.zeros((CB, CATS), f32)
    out_type = (
        jax.ShapeDtypeStruct((B, ITEMS), f32),
        jax.ShapeDtypeStruct((B, CATS), f32),
        jax.ShapeDtypeStruct((B, CATS), f32),
        jax.ShapeDtypeStruct((B, E), f32),
        jax.ShapeDtypeStruct((B, E), f32),
    )
    scratch = [
        pltpu.VMEM((CB, S), jnp.int32),
        pltpu.VMEM((CB, ITEMS), f32),
        pltpu.VMEM((CB, CATS), f32),
        pltpu.VMEM((CB, CATS), f32),
        pltpu.VMEM((ITEMS,), jnp.int32),
        pltpu.VMEM((RW,), jnp.int32),
        pltpu.VMEM((RW,), jnp.int32),
        pltpu.VMEM((NJ, GJ), jnp.int32),
        pltpu.VMEM((NJ, GJ), jnp.int32),
        pltpu.VMEM((GJ, E), f32),
        pltpu.SemaphoreType.DMA,
    ]
    mesh = plsc.VectorSubcoreMesh(core_axis_name="c", subcore_axis_name="s")
    return pl.kernel(_sc_body, out_type=out_type, mesh=mesh,
                     scratch_types=scratch,
                     compiler_params=pltpu.CompilerParams(
                         use_tc_tiling_on_sc=False,
                         needs_layout_passes=False))(
        history_item, item, neg_item, category_list, item_table, zrow, zrow_c)


BM = 1024
NB = B // BM


def _tc_body(ci_ref, ohi_ref, ohn_ref, iti_ref, ngi_ref,
             cl_ref, itab_ref, ctab_ref, wu_ref, bu_ref,
             g_ref, b_ref, w1_ref, b1_ref, w2_ref, b2_ref,
             wo_ref, bo_ref,
             out_ref, hall_ref, tab_ref, st_ref):
    p = pl.program_id(0)
    i = pl.program_id(1)
    f32 = jnp.float32
    hp = lax.Precision.HIGHEST

    @pl.when(p == 0)
    def _phase0():
        @pl.when(i == 0)
        def _():
            m = (cl_ref[...] == lax.broadcasted_iota(jnp.int32,
                                                     (ITEMS, CATS), 1))
            tab = jnp.concatenate(
                [itab_ref[...],
                 lax.dot(m.astype(f32), ctab_ref[...],
                         preferred_element_type=f32, precision=hp)], axis=1)
            t0 = tab.astype(jnp.bfloat16)
            r1 = tab - t0.astype(f32)
            t1 = r1.astype(jnp.bfloat16)
            t2 = (r1 - t1.astype(f32)).astype(jnp.bfloat16)
            tab_ref[0] = t0
            tab_ref[1] = t1
            tab_ref[2] = t2
            st_ref[...] = jnp.zeros_like(st_ref)

        c16 = ci_ref[...].astype(jnp.bfloat16)
        h = (lax.dot(c16, tab_ref[0], preferred_element_type=f32)
             + lax.dot(c16, tab_ref[1], preferred_element_type=f32)
             + lax.dot(c16, tab_ref[2], preferred_element_type=f32)) * (1.0 / S)
        hall_ref[pl.ds(i * BM, BM), :] = h
        st_ref[0:1, :] += jnp.sum(h, axis=0, keepdims=True)
        st_ref[1:2, :] += jnp.sum(h * h, axis=0, keepdims=True)

    @pl.when(p == 1)
    def _phase1():
        inv_b = 1.0 / B
        eps = 1e-5
        mean = st_ref[0:1, :] * inv_b
        var = st_ref[1:2, :] * inv_b - mean * mean
        sc = g_ref[...] * lax.rsqrt(var + eps)
        sh = b_ref[...] - mean * sc

        bn = hall_ref[pl.ds(i * BM, BM), :] * sc + sh
        ue = lax.dot(bn, wu_ref[...]) + bu_ref[...]

        def head(e_i, oh_c):
            e_c = lax.dot(oh_c, ctab_ref[...], preferred_element_type=f32,
                          precision=hp)
            din = jnp.concatenate([ue, e_i, e_c], axis=1)
            t = jnp.maximum(lax.dot(din, w1_ref[...]) + b1_ref[...], 0.0)
            t = jnp.maximum(lax.dot(t, w2_ref[...]) + b2_ref[...], 0.0)
            return lax.dot(t, wo_ref[...]) + bo_ref[...]

        out_ref[...] = (head(iti_ref[...], ohi_ref[...])
                        - head(ngi_ref[...], ohn_ref[...]))


def _tc_head(counts_i, oh_it, oh_ng, it_i, ng_i, category_list,
             item_table, cat_tab100, W_user, b_user, gamma, beta,
             W1, b1, W2, b2, Wout, bout):
    f32 = jnp.float32

    def cmap(bs):
        return pl.BlockSpec(bs, lambda p, i: (i * (1 - p), 0))

    def emap(bs):
        return pl.BlockSpec(bs, lambda p, i: (i * p, 0))

    def wmap(shape):
        return pl.BlockSpec(shape, lambda p, i: (0, 0))

    in_specs = [
        cmap((BM, ITEMS)),
        emap((BM, CATS)), emap((BM, CATS)), emap((BM, E)), emap((BM, E)),
        wmap((ITEMS, 1)), wmap((ITEMS, E)), wmap((CATS, E)),
        wmap((64, 64)), wmap((1, 64)),
        wmap((1, 64)), wmap((1, 64)),
        wmap((128, 200)), wmap((1, 200)),
        wmap((200, 80)), wmap((1, 80)),
        wmap((80, 1)), wmap((1, 1)),
    ]
    return pl.pallas_call(
        _tc_body,
        grid=(2, NB),
        in_specs=in_specs,
        out_specs=pl.BlockSpec((BM, 1), lambda p, i: (i, 0)),
        out_shape=jax.ShapeDtypeStruct((B, 1), f32),
        scratch_shapes=[
            pltpu.VMEM((B, 64), f32),
            pltpu.VMEM((3, ITEMS, 64), jnp.bfloat16),
            pltpu.VMEM((2, 64), f32),
        ],
    )(counts_i, oh_it, oh_ng, it_i, ng_i,
      category_list.reshape(ITEMS, 1), item_table, cat_tab100,
      W_user, b_user.reshape(1, 64),
      gamma.reshape(1, 64), beta.reshape(1, 64),
      W1, b1.reshape(1, 200),
      W2, b2.reshape(1, 80), Wout, bout.reshape(1, 1))


def kernel(user, item, neg_item, history_item, category_list, item_table,
           cat_table, W_user, b_user, gamma, beta, W1, b1, W2, b2, Wout, bout):
    counts_i, oh_it, oh_ng, it_i, ng_i = _sc_counts(
        history_item, item, neg_item, category_list, item_table)
    out = _tc_head(counts_i, oh_it, oh_ng, it_i, ng_i, category_list,
                   item_table, cat_table[:CATS], W_user, b_user, gamma, beta,
                   W1, b1, W2, b2, Wout, bout)
    return out[:, 0]


# speedup vs baseline: 1.2294x; 1.0256x over previous
import jax
import jax.numpy as jnp
from jax import lax
from jax.experimental import pallas as pl
from jax.experimental.pallas import tpu as pltpu
from jax.experimental.pallas import tpu_sc as plsc

B = 16384
S = 200
ITEMS = 1000
CATS = 100
E = 32
NC, NS, L = 2, 16, 16
NW = NC * NS
RW = B // NW
CB = 32
NCHUNK = RW // CB
GJ = 128
NJ = RW // GJ


def _sc_body(hist, item, neg, clist, itab, zrow, zrow_c,
             counts_i, ohic, ohnc, it_i, ng_i,
             hbuf0, hbuf1, cbuf0, cbuf1, ohib0, ohib1, ohnb0, ohnb1,
             catb, itemv, negv, idxb, ngxb, rowb0, rowb1,
             semg, sema, semb):
    wid = lax.axis_index("s") * NC + lax.axis_index("c")
    base = wid * RW
    iota = lax.iota(jnp.int32, L)
    ones = jnp.full((L,), 1.0, jnp.float32)
    neg_ones = jnp.full((L,), -1.0, jnp.float32)
    hbuf = (hbuf0, hbuf1)
    cbuf = (cbuf0, cbuf1)
    ohib = (ohib0, ohib1)
    ohnb = (ohnb0, ohnb1)
    rowb = (rowb0, rowb1)
    sems = (sema, semb)
    rgs = [iota + rg * L for rg in range(CB // L)]

    pltpu.sync_copy(clist, catb)
    pltpu.sync_copy(item.at[pl.ds(base, RW)], itemv)
    pltpu.sync_copy(neg.at[pl.ds(base, RW)], negv)
    for j in range(NJ):
        pltpu.sync_copy(item.at[pl.ds(base + j * GJ, GJ)], idxb.at[j])
        pltpu.sync_copy(neg.at[pl.ds(base + j * GJ, GJ)], ngxb.at[j])

    seq = [(idxb, it_i, j) for j in range(NJ)] + \
          [(ngxb, ng_i, j) for j in range(NJ)]
    gd = {}
    od = {}
    for t, (idxr, out, j) in enumerate(seq):
        p = t % 2
        if t >= 2:
            od[p].wait()
        gd[p] = pltpu.async_copy(itab.at[idxr.at[j]], rowb[p], semg)
        if t >= 1:
            q = 1 - p
            gd[q].wait()
            pidxr, pout, pj = seq[t - 1]
            od[q] = pltpu.async_copy(
                rowb[q], pout.at[pl.ds(base + pj * GJ, GJ)], sems[q])
    lp = (len(seq) - 1) % 2
    gd[lp].wait()
    lidxr, lout, lj = seq[-1]
    od[lp] = pltpu.async_copy(
        rowb[lp], lout.at[pl.ds(base + lj * GJ, GJ)], sems[lp])
    od[0].wait()
    od[1].wait()

    pltpu.sync_copy(zrow, cbuf0)
    pltpu.sync_copy(zrow, cbuf1)
    pltpu.sync_copy(zrow_c, ohib0)
    pltpu.sync_copy(zrow_c, ohib1)
    pltpu.sync_copy(zrow_c, ohnb0)
    pltpu.sync_copy(zrow_c, ohnb1)

    def hist_pass(cb_ref, hb_ref, val):
        def sbody(k, sv):
            for _ in range(4):
                hs = [plsc.load_gather(hb_ref, [rows, sv]) for rows in rgs]
                for rows, h in zip(rgs, hs):
                    plsc.addupdate_scatter(cb_ref, [rows, h], val)
                sv = sv + 1
            return sv

        lax.fori_loop(0, S // 4, sbody, jnp.zeros((L,), jnp.int32))

    def oh_pass(oi_ref, on_ref, c, val):
        for rg in range(CB // L):
            rows = iota + rg * L
            iv = itemv[pl.ds(c * CB + rg * L, L)]
            plsc.addupdate_scatter(oi_ref, [rows, plsc.load_gather(catb, [iv])],
                                   val)
            nv = negv[pl.ds(c * CB + rg * L, L)]
            plsc.addupdate_scatter(on_ref, [rows, plsc.load_gather(catb, [nv])],
                                   val)

    outd = {}
    for c in range(NCHUNK):
        p = c % 2
        if c >= 2:
            for d in outd[p]:
                d.wait()
            hist_pass(cbuf[p], hbuf[p], neg_ones)
            oh_pass(ohib[p], ohnb[p], c - 2, neg_ones)
        cb = base + c * CB
        pltpu.sync_copy(hist.at[pl.ds(cb, CB)], hbuf[p])
        hist_pass(cbuf[p], hbuf[p], ones)
        oh_pass(ohib[p], ohnb[p], c, ones)
        outd[p] = (
            pltpu.async_copy(cbuf[p], counts_i.at[pl.ds(cb, CB)], sems[p]),
            pltpu.async_copy(ohib[p], ohic.at[pl.ds(cb, CB)], sems[p]),
            pltpu.async_copy(ohnb[p], ohnc.at[pl.ds(cb, CB)], sems[p]),
        )
    for p in (0, 1):
        for d in outd[p]:
            d.wait()


def _sc_counts(history_item, item, neg_item, category_list, item_table):
    f32 = jnp.float32
    zrow = jnp.zeros((CB, ITEMS), f32)
    zrow_c = jnp.zeros((CB, CATS), f32)
    out_type = (
        jax.ShapeDtypeStruct((B, ITEMS), f32),
        jax.ShapeDtypeStruct((B, CATS), f32),
        jax.ShapeDtypeStruct((B, CATS), f32),
        jax.ShapeDtypeStruct((B, E), f32),
        jax.ShapeDtypeStruct((B, E), f32),
    )
    scratch = [
        pltpu.VMEM((CB, S), jnp.int32),
        pltpu.VMEM((CB, S), jnp.int32),
        pltpu.VMEM((CB, ITEMS), f32),
        pltpu.VMEM((CB, ITEMS), f32),
        pltpu.VMEM((CB, CATS), f32),
        pltpu.VMEM((CB, CATS), f32),
        pltpu.VMEM((CB, CATS), f32),
        pltpu.VMEM((CB, CATS), f32),
        pltpu.VMEM((ITEMS,), jnp.int32),
        pltpu.VMEM((RW,), jnp.int32),
        pltpu.VMEM((RW,), jnp.int32),
        pltpu.VMEM((NJ, GJ), jnp.int32),
        pltpu.VMEM((NJ, GJ), jnp.int32),
        pltpu.VMEM((GJ, E), f32),
        pltpu.VMEM((GJ, E), f32),
        pltpu.SemaphoreType.DMA,
        pltpu.SemaphoreType.DMA,
        pltpu.SemaphoreType.DMA,
    ]
    mesh = plsc.VectorSubcoreMesh(core_axis_name="c", subcore_axis_name="s")
    return pl.kernel(_sc_body, out_type=out_type, mesh=mesh,
                     scratch_types=scratch,
                     compiler_params=pltpu.CompilerParams(
                         use_tc_tiling_on_sc=False,
                         needs_layout_passes=False))(
        history_item, item, neg_item, category_list, item_table, zrow, zrow_c)


BM = 1024
NB = B // BM


def _tc_body(ci_ref, ohi_ref, ohn_ref, iti_ref, ngi_ref,
             cl_ref, itab_ref, ctab_ref, wu_ref, bu_ref,
             g_ref, b_ref, w1_ref, b1_ref, w2_ref, b2_ref,
             wo_ref, bo_ref,
             out_ref, hall_ref, tab_ref, st_ref):
    p = pl.program_id(0)
    i = pl.program_id(1)
    f32 = jnp.float32
    hp = lax.Precision.HIGHEST

    @pl.when(p == 0)
    def _phase0():
        @pl.when(i == 0)
        def _():
            m = (cl_ref[...] == lax.broadcasted_iota(jnp.int32,
                                                     (ITEMS, CATS), 1))
            tab = jnp.concatenate(
                [itab_ref[...],
                 lax.dot(m.astype(f32), ctab_ref[...],
                         preferred_element_type=f32, precision=hp)], axis=1)
            t0 = tab.astype(jnp.bfloat16)
            r1 = tab - t0.astype(f32)
            t1 = r1.astype(jnp.bfloat16)
            t2 = (r1 - t1.astype(f32)).astype(jnp.bfloat16)
            tab_ref[0] = t0
            tab_ref[1] = t1
            tab_ref[2] = t2
            st_ref[...] = jnp.zeros_like(st_ref)

        c16 = ci_ref[...].astype(jnp.bfloat16)
        h = (lax.dot(c16, tab_ref[0], preferred_element_type=f32)
             + lax.dot(c16, tab_ref[1], preferred_element_type=f32)
             + lax.dot(c16, tab_ref[2], preferred_element_type=f32)) * (1.0 / S)
        hall_ref[pl.ds(i * BM, BM), :] = h
        st_ref[0:1, :] += jnp.sum(h, axis=0, keepdims=True)
        st_ref[1:2, :] += jnp.sum(h * h, axis=0, keepdims=True)

    @pl.when(p == 1)
    def _phase1():
        inv_b = 1.0 / B
        eps = 1e-5
        mean = st_ref[0:1, :] * inv_b
        var = st_ref[1:2, :] * inv_b - mean * mean
        sc = g_ref[...] * lax.rsqrt(var + eps)
        sh = b_ref[...] - mean * sc

        bn = hall_ref[pl.ds(i * BM, BM), :] * sc + sh
        ue = lax.dot(bn, wu_ref[...]) + bu_ref[...]

        def head(e_i, oh_c):
            e_c = lax.dot(oh_c, ctab_ref[...], preferred_element_type=f32,
                          precision=hp)
            din = jnp.concatenate([ue, e_i, e_c], axis=1)
            t = jnp.maximum(lax.dot(din, w1_ref[...]) + b1_ref[...], 0.0)
            t = jnp.maximum(lax.dot(t, w2_ref[...]) + b2_ref[...], 0.0)
            return lax.dot(t, wo_ref[...]) + bo_ref[...]

        out_ref[...] = (head(iti_ref[...], ohi_ref[...])
                        - head(ngi_ref[...], ohn_ref[...]))


def _tc_head(counts_i, oh_it, oh_ng, it_i, ng_i, category_list,
             item_table, cat_tab100, W_user, b_user, gamma, beta,
             W1, b1, W2, b2, Wout, bout):
    f32 = jnp.float32

    def cmap(bs):
        return pl.BlockSpec(bs, lambda p, i: (i * (1 - p), 0))

    def emap(bs):
        return pl.BlockSpec(bs, lambda p, i: (i * p, 0))

    def wmap(shape):
        return pl.BlockSpec(shape, lambda p, i: (0, 0))

    in_specs = [
        cmap((BM, ITEMS)),
        emap((BM, CATS)), emap((BM, CATS)), emap((BM, E)), emap((BM, E)),
        wmap((ITEMS, 1)), wmap((ITEMS, E)), wmap((CATS, E)),
        wmap((64, 64)), wmap((1, 64)),
        wmap((1, 64)), wmap((1, 64)),
        wmap((128, 200)), wmap((1, 200)),
        wmap((200, 80)), wmap((1, 80)),
        wmap((80, 1)), wmap((1, 1)),
    ]
    return pl.pallas_call(
        _tc_body,
        grid=(2, NB),
        in_specs=in_specs,
        out_specs=pl.BlockSpec((BM, 1), lambda p, i: (i, 0)),
        out_shape=jax.ShapeDtypeStruct((B, 1), f32),
        scratch_shapes=[
            pltpu.VMEM((B, 64), f32),
            pltpu.VMEM((3, ITEMS, 64), jnp.bfloat16),
            pltpu.VMEM((2, 64), f32),
        ],
    )(counts_i, oh_it, oh_ng, it_i, ng_i,
      category_list.reshape(ITEMS, 1), item_table, cat_tab100,
      W_user, b_user.reshape(1, 64),
      gamma.reshape(1, 64), beta.reshape(1, 64),
      W1, b1.reshape(1, 200),
      W2, b2.reshape(1, 80), Wout, bout.reshape(1, 1))


def kernel(user, item, neg_item, history_item, category_list, item_table,
           cat_table, W_user, b_user, gamma, beta, W1, b1, W2, b2, Wout, bout):
    counts_i, oh_it, oh_ng, it_i, ng_i = _sc_counts(
        history_item, item, neg_item, category_list, item_table)
    out = _tc_head(counts_i, oh_it, oh_ng, it_i, ng_i, category_list,
                   item_table, cat_table[:CATS], W_user, b_user, gamma, beta,
                   W1, b1, W2, b2, Wout, bout)
    return out[:, 0]
